# baseline (speedup 1.0000x reference)
import functools

import jax
import jax.numpy as jnp
from jax import lax
from jax.experimental import pallas as pl
from jax.experimental.pallas import tpu as pltpu
from jax.experimental.pallas import tpu_sc as plsc

_DIM = 64
_NC = 2
_NS = 16
_NW = _NC * _NS
_RPG = 128
_B = 4096 * 50
_PER_W = _B // _NW
_NJ = _PER_W // _RPG

_mesh = plsc.VectorSubcoreMesh(core_axis_name="c", subcore_axis_name="s")


@functools.partial(
    pl.kernel,
    mesh=_mesh,
    out_type=jax.ShapeDtypeStruct((_NW * _NJ, _RPG, _DIM), jnp.float32),
    scratch_types=[
        pltpu.VMEM((_NJ, _RPG), jnp.int32),
        pltpu.VMEM((_RPG, _DIM), jnp.float32),
        pltpu.SemaphoreType.DMA,
    ],
    compiler_params=pltpu.CompilerParams(use_tc_tiling_on_sc=False),
)
def _gather(x_hbm, table_hbm, out_hbm, idx_v, rows_v, sem):
    wid = lax.axis_index("s") * _NC + lax.axis_index("c")
    pltpu.sync_copy(x_hbm.at[wid], idx_v)

    def body(j, carry):
        pltpu.async_copy(table_hbm.at[idx_v.at[j]], rows_v, sem).wait()
        pltpu.sync_copy(rows_v, out_hbm.at[wid * _NJ + j])
        return carry

    lax.fori_loop(0, _NJ, body, 0)


def kernel(x, table):
    xi = x.reshape(_NW, _NJ, _RPG).astype(jnp.int32)
    out = _gather(xi, table)
    return out.reshape(4096, 50, _DIM)


# speedup vs baseline: 1.0462x; 1.0462x over previous
import functools

import jax
import jax.numpy as jnp
from jax import lax
from jax.experimental import pallas as pl
from jax.experimental.pallas import tpu as pltpu
from jax.experimental.pallas import tpu_sc as plsc

_DIM = 64
_NC = 2
_NS = 16
_NW = _NC * _NS
_RPG = 128
_B = 4096 * 50
_PER_W = _B // _NW
_NJ = _PER_W // _RPG
_NBUF = 5
_NOUT = _NJ // _NBUF

_mesh = plsc.VectorSubcoreMesh(core_axis_name="c", subcore_axis_name="s")


@functools.partial(
    pl.kernel,
    mesh=_mesh,
    out_type=jax.ShapeDtypeStruct((_NW * _NJ, _RPG, _DIM), jnp.float32),
    scratch_types=[
        pltpu.VMEM((_NJ, _RPG), jnp.int32),
        pltpu.VMEM((_NBUF, _RPG, _DIM), jnp.float32),
        pltpu.SemaphoreType.DMA((_NBUF,)),
        pltpu.SemaphoreType.DMA((_NBUF,)),
    ],
    compiler_params=pltpu.CompilerParams(use_tc_tiling_on_sc=False),
)
def _gather(x_hbm, table_hbm, out_hbm, idx_v, rows_v, gsem, ssem):
    wid = lax.axis_index("s") * _NC + lax.axis_index("c")
    pltpu.sync_copy(x_hbm.at[wid], idx_v)

    for b in range(_NBUF):
        pltpu.async_copy(table_hbm.at[idx_v.at[b]], rows_v.at[b], gsem.at[b])

    def body(i, carry):
        base = i * _NBUF
        for b in range(_NBUF):
            j = base + b
            pltpu.make_async_copy(
                table_hbm.at[idx_v.at[j]], rows_v.at[b], gsem.at[b]
            ).wait()
            pltpu.async_copy(rows_v.at[b], out_hbm.at[wid * _NJ + j], ssem.at[b])

        @pl.when(i < _NOUT - 1)
        def _():
            for b in range(_NBUF):
                j2 = base + _NBUF + b
                pltpu.make_async_copy(
                    rows_v.at[b], out_hbm.at[wid * _NJ + base + b], ssem.at[b]
                ).wait()
                pltpu.async_copy(table_hbm.at[idx_v.at[j2]], rows_v.at[b], gsem.at[b])

        return carry

    lax.fori_loop(0, _NOUT, body, 0)

    base = (_NOUT - 1) * _NBUF
    for b in range(_NBUF):
        pltpu.make_async_copy(
            rows_v.at[b], out_hbm.at[wid * _NJ + base + b], ssem.at[b]
        ).wait()


def kernel(x, table):
    xi = x.reshape(_NW, _NJ, _RPG).astype(jnp.int32)
    out = _gather(xi, table)
    return out.reshape(4096, 50, _DIM)
